# baseline (speedup 1.0000x reference)
import jax
import jax.numpy as jnp
from jax.experimental import pallas as pl
from jax.experimental.pallas import tpu as pltpu

_B, _S, _DIM = 2, 2048, 1024
_E, _TOPK, _HID = 8, 2, 64
_EH = _E * _HID
_T = _B * _S
_TM = 512
_NBLK = _T // _TM

_DN_T = (((1,), (1,)), ((), ()))
_DN = (((1,), (0,)), ((), ()))


def _moe_block_kernel(x_ref, gate_w_ref, w1r_ref, w3r_ref, w2c_ref,
                      cls_w_ref, cls_b_ref, out_ref, logits_ref,
                      w1c_ref, w3c_ref):
    i = pl.program_id(0)

    @pl.when(i == 0)
    def _prep():
        w1c_ref[...] = w1r_ref[...].astype(jnp.bfloat16)
        w3c_ref[...] = w3r_ref[...].astype(jnp.bfloat16)

    xb = x_ref[...]
    xb16 = xb.astype(jnp.bfloat16)

    gl = jax.lax.dot_general(xb, gate_w_ref[...], _DN_T,
                             preferred_element_type=jnp.float32)
    eio = jax.lax.broadcasted_iota(jnp.int32, (_TM, _E), 1)
    m1 = jnp.max(gl, axis=-1, keepdims=True)
    i1 = jnp.min(jnp.where(gl >= m1, eio, _E), axis=-1, keepdims=True)
    pm = jnp.where(eio == i1, -jnp.inf, gl)
    m2 = jnp.max(pm, axis=-1, keepdims=True)
    i2 = jnp.min(jnp.where(pm >= m2, eio, _E), axis=-1, keepdims=True)
    a1 = jax.nn.sigmoid(m1 - m2)
    a2 = 1.0 - a1

    h1 = jax.lax.dot_general(xb16, w1c_ref[...], _DN_T,
                             preferred_element_type=jnp.float32)
    h3 = jax.lax.dot_general(xb16, w3c_ref[...], _DN_T,
                             preferred_element_type=jnp.float32)
    h = (h1 * jax.nn.sigmoid(h1)) * h3
    lane_e = jax.lax.broadcasted_iota(jnp.int32, (_TM, _EH), 1) // _HID
    w_exp = (jnp.where(lane_e == i1, a1, 0.0)
             + jnp.where(lane_e == i2, a2, 0.0))
    hw = (h * w_exp).astype(jnp.bfloat16)
    out_ref[...] = jax.lax.dot_general(hw, w2c_ref[...], _DN_T,
                                       preferred_element_type=jnp.float32)

    @pl.when(jnp.logical_or(i == 0, i == _S // _TM))
    def _classifier():
        v = jax.lax.dot_general(cls_w_ref[...].astype(jnp.bfloat16),
                                w2c_ref[...], _DN,
                                preferred_element_type=jnp.float32)
        u = hw[0:1, :].astype(jnp.float32) * v
        lio = jax.lax.broadcasted_iota(jnp.int32, (_EH, _E), 0) // _HID
        ecol = jax.lax.broadcasted_iota(jnp.int32, (_EH, _E), 1)
        mcum = (lio <= ecol).astype(jnp.float32)
        lg = jax.lax.dot_general(u, mcum, _DN,
                                 preferred_element_type=jnp.float32)
        logits_ref[...] = (lg + cls_b_ref[...]).reshape(1, 1, _E)


def kernel(x, tgt_pad, gate_w, cls_w, cls_b, w1, w2, w3):
    del tgt_pad
    xf = x.reshape(_T, _DIM)
    w1r = w1.reshape(_EH, _DIM)
    w3r = w3.reshape(_EH, _DIM)
    w2c = jnp.transpose(w2, (1, 0, 2)).reshape(_DIM, _EH).astype(jnp.bfloat16)
    cls_b2 = cls_b.reshape(1, 1)

    out, logits = pl.pallas_call(
        _moe_block_kernel,
        grid=(_NBLK,),
        in_specs=[
            pl.BlockSpec((_TM, _DIM), lambda i: (i, 0)),
            pl.BlockSpec((_E, _DIM), lambda i: (0, 0)),
            pl.BlockSpec((_EH, _DIM), lambda i: (0, 0)),
            pl.BlockSpec((_EH, _DIM), lambda i: (0, 0)),
            pl.BlockSpec((_DIM, _EH), lambda i: (0, 0)),
            pl.BlockSpec((1, _DIM), lambda i: (0, 0)),
            pl.BlockSpec((1, 1), lambda i: (0, 0)),
        ],
        out_specs=[
            pl.BlockSpec((_TM, _DIM), lambda i: (i, 0)),
            pl.BlockSpec((1, 1, _E), lambda i: (i * _TM // _S, 0, 0)),
        ],
        out_shape=[
            jax.ShapeDtypeStruct((_T, _DIM), jnp.float32),
            jax.ShapeDtypeStruct((_B, 1, _E), jnp.float32),
        ],
        scratch_shapes=[
            pltpu.VMEM((_EH, _DIM), jnp.bfloat16),
            pltpu.VMEM((_EH, _DIM), jnp.bfloat16),
        ],
    )(xf, gate_w, w1r, w3r, w2c, cls_w, cls_b2)

    final_hidden_states = out.reshape(_B, _S, _DIM)
    expert_logits = logits[:, 0, :].T.reshape(_E, _B, 1)
    return final_hidden_states, expert_logits


# speedup vs baseline: 1.0177x; 1.0177x over previous
import jax
import jax.numpy as jnp
from jax.experimental import pallas as pl
from jax.experimental.pallas import tpu as pltpu

_B, _S, _DIM = 2, 2048, 1024
_E, _TOPK, _HID = 8, 2, 64
_EH = _E * _HID
_T = _B * _S
_TM = 1024
_NBLK = _T // _TM

_DN_T = (((1,), (1,)), ((), ()))
_DN = (((1,), (0,)), ((), ()))


def _moe_block_kernel(x_ref, gate_w_ref, w1r_ref, w3r_ref, w2c_ref,
                      cls_w_ref, cls_b_ref, out_ref, logits_ref,
                      w1c_ref, w3c_ref):
    i = pl.program_id(0)

    @pl.when(i == 0)
    def _prep():
        w1c_ref[...] = w1r_ref[...].astype(jnp.bfloat16)
        w3c_ref[...] = w3r_ref[...].astype(jnp.bfloat16)

    xb = x_ref[...]
    xb16 = xb.astype(jnp.bfloat16)

    gl = jax.lax.dot_general(xb, gate_w_ref[...], _DN_T,
                             preferred_element_type=jnp.float32)
    eio = jax.lax.broadcasted_iota(jnp.int32, (_TM, _E), 1)
    m1 = jnp.max(gl, axis=-1, keepdims=True)
    i1 = jnp.min(jnp.where(gl >= m1, eio, _E), axis=-1, keepdims=True)
    pm = jnp.where(eio == i1, -jnp.inf, gl)
    m2 = jnp.max(pm, axis=-1, keepdims=True)
    i2 = jnp.min(jnp.where(pm >= m2, eio, _E), axis=-1, keepdims=True)
    a1 = jax.nn.sigmoid(m1 - m2)
    a2 = 1.0 - a1

    h1 = jax.lax.dot_general(xb16, w1c_ref[...], _DN_T,
                             preferred_element_type=jnp.float32)
    h3 = jax.lax.dot_general(xb16, w3c_ref[...], _DN_T,
                             preferred_element_type=jnp.float32)
    h = (h1 * jax.nn.sigmoid(h1)) * h3
    lane_e = jax.lax.broadcasted_iota(jnp.int32, (_TM, _EH), 1) // _HID
    w_exp = (jnp.where(lane_e == i1, a1, 0.0)
             + jnp.where(lane_e == i2, a2, 0.0))
    hw = (h * w_exp).astype(jnp.bfloat16)
    out_ref[...] = jax.lax.dot_general(hw, w2c_ref[...], _DN_T,
                                       preferred_element_type=jnp.float32)

    @pl.when(jnp.logical_or(i == 0, i == _S // _TM))
    def _classifier():
        v = jax.lax.dot_general(cls_w_ref[...].astype(jnp.bfloat16),
                                w2c_ref[...], _DN,
                                preferred_element_type=jnp.float32)
        u = hw[0:1, :].astype(jnp.float32) * v
        lio = jax.lax.broadcasted_iota(jnp.int32, (_EH, _E), 0) // _HID
        ecol = jax.lax.broadcasted_iota(jnp.int32, (_EH, _E), 1)
        mcum = (lio <= ecol).astype(jnp.float32)
        lg = jax.lax.dot_general(u, mcum, _DN,
                                 preferred_element_type=jnp.float32)
        logits_ref[...] = (lg + cls_b_ref[...]).reshape(1, 1, _E)


def kernel(x, tgt_pad, gate_w, cls_w, cls_b, w1, w2, w3):
    del tgt_pad
    xf = x.reshape(_T, _DIM)
    w1r = w1.reshape(_EH, _DIM)
    w3r = w3.reshape(_EH, _DIM)
    w2c = jnp.transpose(w2, (1, 0, 2)).reshape(_DIM, _EH).astype(jnp.bfloat16)
    cls_b2 = cls_b.reshape(1, 1)

    out, logits = pl.pallas_call(
        _moe_block_kernel,
        grid=(_NBLK,),
        in_specs=[
            pl.BlockSpec((_TM, _DIM), lambda i: (i, 0)),
            pl.BlockSpec((_E, _DIM), lambda i: (0, 0)),
            pl.BlockSpec((_EH, _DIM), lambda i: (0, 0)),
            pl.BlockSpec((_EH, _DIM), lambda i: (0, 0)),
            pl.BlockSpec((_DIM, _EH), lambda i: (0, 0)),
            pl.BlockSpec((1, _DIM), lambda i: (0, 0)),
            pl.BlockSpec((1, 1), lambda i: (0, 0)),
        ],
        out_specs=[
            pl.BlockSpec((_TM, _DIM), lambda i: (i, 0)),
            pl.BlockSpec((1, 1, _E), lambda i: (i * _TM // _S, 0, 0)),
        ],
        out_shape=[
            jax.ShapeDtypeStruct((_T, _DIM), jnp.float32),
            jax.ShapeDtypeStruct((_B, 1, _E), jnp.float32),
        ],
        scratch_shapes=[
            pltpu.VMEM((_EH, _DIM), jnp.bfloat16),
            pltpu.VMEM((_EH, _DIM), jnp.bfloat16),
        ],
    )(xf, gate_w, w1r, w3r, w2c, cls_w, cls_b2)

    final_hidden_states = out.reshape(_B, _S, _DIM)
    expert_logits = logits[:, 0, :].T.reshape(_E, _B, 1)
    return final_hidden_states, expert_logits
